# Initial kernel scaffold; baseline (speedup 1.0000x reference)
import jax
import jax.numpy as jnp
from jax.experimental import pallas as pl


def kernel(x, edge_index, batch, edge_attr, node_W, node_b, edge_W, edge_b, lin_W, lin_b, W1, b1, gamma, beta, W2, b2, out_W, out_b):
    raise NotImplementedError("write your pallas kernel here")


# speedup vs baseline: 2.3613x; 2.3613x over previous
import functools

import jax
import jax.numpy as jnp
from jax import lax
from jax.experimental import pallas as pl
from jax.experimental.pallas import tpu as pltpu
from jax.experimental.pallas import tpu_sc as plsc

NC = 2
NS = 16
NW = NC * NS
EB = 80
GOUT = 64


def _h0_body(x_ref, w_ref, b_ref, o_ref):
    o_ref[...] = (
        jnp.dot(x_ref[...], w_ref[...], preferred_element_type=jnp.float32)
        + b_ref[...]
    )


def _h0(x, node_W, node_b):
    n, d = x.shape
    h = node_W.shape[1]
    nb = 10
    return pl.pallas_call(
        _h0_body,
        grid=(nb,),
        in_specs=[
            pl.BlockSpec((n // nb, d), lambda i: (i, 0)),
            pl.BlockSpec((d, h), lambda i: (0, 0)),
            pl.BlockSpec((1, h), lambda i: (0, 0)),
        ],
        out_specs=pl.BlockSpec((n // nb, h), lambda i: (i, 0)),
        out_shape=jax.ShapeDtypeStruct((n, h), jnp.float32),
    )(x, node_W, node_b.reshape(1, h))


def _fold_body(ew_ref, eb_ref, lw_ref, lb_ref, m_ref, c_ref):
    nl = lw_ref.shape[0]
    for i in range(nl):
        m_ref[i] = jnp.dot(ew_ref[...], lw_ref[i], preferred_element_type=jnp.float32)
        c_ref[i] = (
            jnp.dot(eb_ref[...], lw_ref[i], preferred_element_type=jnp.float32)[0]
            + lb_ref[i]
        )


def _fold(edge_W, edge_b, lin_W, lin_b):
    de, h = edge_W.shape
    nl = lin_W.shape[0]
    return pl.pallas_call(
        _fold_body,
        out_shape=[
            jax.ShapeDtypeStruct((nl, de, h), jnp.float32),
            jax.ShapeDtypeStruct((nl, h), jnp.float32),
        ],
    )(edge_W, edge_b.reshape(1, h), lin_W, lin_b)


def _emsg_body(ea_ref, big_ref, c_ref, o_ref):
    o_ref[...] = (
        jnp.dot(ea_ref[...], big_ref[...], preferred_element_type=jnp.float32)
        + c_ref[...]
    )


def _emsg(ea_packed, bigM, bigc):
    e8, k = ea_packed.shape
    kh = bigM.shape[1]
    nb = 50
    return pl.pallas_call(
        _emsg_body,
        grid=(nb,),
        in_specs=[
            pl.BlockSpec((e8 // nb, k), lambda i: (i, 0)),
            pl.BlockSpec((k, kh), lambda i: (0, 0)),
            pl.BlockSpec((1, kh), lambda i: (0, 0)),
        ],
        out_specs=pl.BlockSpec((e8 // nb, kh), lambda i: (i, 0)),
        out_shape=jax.ShapeDtypeStruct((e8, kh), jnp.float32),
    )(ea_packed, bigM, bigc)


def _sc_body(n, h, epw, nblk, rpt, zr,
             h_hbm, e_hbm, src_hbm, dst_hbm, out_hbm,
             sidx, didx, ebuf, gbuf, zbuf, agg):
    c = lax.axis_index("c")
    s = lax.axis_index("s")
    nv = h // 16
    tail = n - NS * rpt

    zero = jnp.zeros((16,), jnp.float32)

    def zrow(r, carry):
        for k in range(nv):
            zbuf[r, pl.ds(k * 16, 16)] = zero
        return carry

    lax.fori_loop(0, zr, zrow, 0)
    row0 = s * rpt
    for t in range(rpt // zr):
        pltpu.sync_copy(zbuf, agg.at[pl.ds(row0 + t * zr, zr)])
    if tail:
        @pl.when(s == 0)
        def _():
            pltpu.sync_copy(zbuf.at[pl.ds(0, tail)], agg.at[pl.ds(NS * rpt, tail)])
    plsc.subcore_barrier()

    ebase = (c * NS + s) * epw

    def step(j, carry):
        eb = ebase + j * EB
        pltpu.sync_copy(src_hbm.at[pl.ds(eb, EB)], sidx)
        pltpu.sync_copy(dst_hbm.at[pl.ds(eb, EB)], didx)
        pltpu.sync_copy(e_hbm.at[pl.ds(eb, EB)], ebuf)
        pltpu.sync_copy(h_hbm.at[sidx], gbuf)

        def relu_row(r, carry2):
            for k in range(nv):
                sl = pl.ds(k * 16, 16)
                ebuf[r, sl] = jnp.maximum(ebuf[r, sl] + gbuf[r, sl], 0.0)
            return carry2

        lax.fori_loop(0, EB, relu_row, 0)
        pltpu.sync_copy(ebuf, agg.at[didx], add=True)
        return carry

    lax.fori_loop(0, nblk, step, 0)
    plsc.subcore_barrier()
    pltpu.sync_copy(agg.at[pl.ds(row0, rpt)], out_hbm.at[c, pl.ds(row0, rpt)])
    if tail:
        @pl.when(s == 0)
        def _():
            pltpu.sync_copy(agg.at[pl.ds(NS * rpt, tail)],
                            out_hbm.at[c, pl.ds(NS * rpt, tail)])


def _sc_agg(h_nodes, e_msg, src, dst):
    n, h = h_nodes.shape
    e = src.shape[0]
    epw = e // NW
    nblk = epw // EB
    rpt = (n // NS) // 8 * 8
    zr = 104

    mesh = plsc.VectorSubcoreMesh(core_axis_name="c", subcore_axis_name="s")
    body = functools.partial(_sc_body, n, h, epw, nblk, rpt, zr)
    return pl.kernel(
        body,
        out_type=jax.ShapeDtypeStruct((NC, n, h), jnp.float32),
        mesh=mesh,
        scratch_types=[
            pltpu.VMEM((EB,), jnp.int32),
            pltpu.VMEM((EB,), jnp.int32),
            pltpu.VMEM((EB, h), jnp.float32),
            pltpu.VMEM((EB, h), jnp.float32),
            pltpu.VMEM((zr, h), jnp.float32),
            pltpu.VMEM_SHARED((n, h), jnp.float32),
        ],
    )(h_nodes, e_msg, src, dst)


def _mlp1_body(h_ref, agg_ref, w_ref, b_ref, z_ref, st_ref):
    i = pl.program_id(0)
    z = h_ref[...] + agg_ref[0] + agg_ref[1]
    z1 = jnp.dot(z, w_ref[...], preferred_element_type=jnp.float32) + b_ref[...]
    z_ref[...] = z1
    ps = jnp.sum(z1, axis=0, keepdims=True)
    pss = jnp.sum(z1 * z1, axis=0, keepdims=True)
    blk = jnp.concatenate([ps, pss], axis=0)

    @pl.when(i == 0)
    def _():
        st_ref[...] = blk

    @pl.when(i != 0)
    def _():
        st_ref[...] = st_ref[...] + blk


def _mlp1(h_nodes, agg, W1, b1):
    n, h = h_nodes.shape
    nb = 10
    return pl.pallas_call(
        _mlp1_body,
        grid=(nb,),
        in_specs=[
            pl.BlockSpec((n // nb, h), lambda i: (i, 0)),
            pl.BlockSpec((NC, n // nb, h), lambda i: (0, i, 0)),
            pl.BlockSpec((h, h), lambda i: (0, 0)),
            pl.BlockSpec((1, h), lambda i: (0, 0)),
        ],
        out_specs=[
            pl.BlockSpec((n // nb, h), lambda i: (i, 0)),
            pl.BlockSpec((2, h), lambda i: (0, 0)),
        ],
        out_shape=[
            jax.ShapeDtypeStruct((n, h), jnp.float32),
            jax.ShapeDtypeStruct((2, h), jnp.float32),
        ],
    )(h_nodes, agg, W1, b1.reshape(1, h))


def _mlp2_body(n, z_ref, st_ref, g_ref, be_ref, w_ref, b_ref, o_ref):
    mean = st_ref[0] / n
    var = st_ref[1] / n - mean * mean
    inv = g_ref[0] * lax.rsqrt(var + 1e-5)
    xn = (z_ref[...] - mean) * inv + be_ref[0]
    a = jnp.maximum(xn, 0.0)
    z2 = jnp.dot(a, w_ref[...], preferred_element_type=jnp.float32) + b_ref[...]
    o_ref[...] = jnp.maximum(z2, 0.0)


def _mlp2(z1, stats, gamma, beta, W2, b2):
    n, h = z1.shape
    nb = 10
    body = functools.partial(_mlp2_body, float(n))
    return pl.pallas_call(
        body,
        grid=(nb,),
        in_specs=[
            pl.BlockSpec((n // nb, h), lambda i: (i, 0)),
            pl.BlockSpec((2, h), lambda i: (0, 0)),
            pl.BlockSpec((1, h), lambda i: (0, 0)),
            pl.BlockSpec((1, h), lambda i: (0, 0)),
            pl.BlockSpec((h, h), lambda i: (0, 0)),
            pl.BlockSpec((1, h), lambda i: (0, 0)),
        ],
        out_specs=pl.BlockSpec((n // nb, h), lambda i: (i, 0)),
        out_shape=jax.ShapeDtypeStruct((n, h), jnp.float32),
    )(z1, stats, gamma.reshape(1, h), beta.reshape(1, h), W2, b2.reshape(1, h))


def _pool_body(nb, h_ref, bat_ref, w_ref, b_ref, o_ref, acc):
    i = pl.program_id(0)
    b = bat_ref[0, 0, :]
    rows = b.shape[0]
    oh = (
        lax.broadcasted_iota(jnp.int32, (GOUT, rows), 0) == b[None, :]
    ).astype(jnp.float32)
    part = jnp.dot(oh, h_ref[...], preferred_element_type=jnp.float32)

    @pl.when(i == 0)
    def _():
        acc[...] = part

    @pl.when(i != 0)
    def _():
        acc[...] = acc[...] + part

    @pl.when(i == nb - 1)
    def _():
        o_ref[...] = (
            jnp.dot(acc[...], w_ref[...], preferred_element_type=jnp.float32)
            + b_ref[...]
        )


def _pool(h_nodes, batch3, out_Wp, out_bp):
    n, h = h_nodes.shape
    nb = 10
    cw = out_Wp.shape[1]
    body = functools.partial(_pool_body, nb)
    return pl.pallas_call(
        body,
        grid=(nb,),
        in_specs=[
            pl.BlockSpec((n // nb, h), lambda i: (i, 0)),
            pl.BlockSpec((1, 1, n // nb), lambda i: (i, 0, 0)),
            pl.BlockSpec((h, cw), lambda i: (0, 0)),
            pl.BlockSpec((1, cw), lambda i: (0, 0)),
        ],
        out_specs=pl.BlockSpec((GOUT, cw), lambda i: (0, 0)),
        out_shape=jax.ShapeDtypeStruct((GOUT, cw), jnp.float32),
        scratch_shapes=[pltpu.VMEM((GOUT, h), jnp.float32)],
    )(h_nodes, batch3, out_Wp, out_bp)


def kernel(x, edge_index, batch, edge_attr, node_W, node_b, edge_W, edge_b,
           lin_W, lin_b, W1, b1, gamma, beta, W2, b2, out_W, out_b):
    n, d = x.shape
    e = edge_index.shape[1]
    h = node_W.shape[1]
    de = edge_attr.shape[1]
    nl = lin_W.shape[0]
    c = out_W.shape[1]

    src = edge_index[0]
    dst = edge_index[1]

    h0 = _h0(x, node_W, node_b)
    M, cvec = _fold(edge_W, edge_b, lin_W, lin_b)

    eye8 = jnp.eye(8, dtype=jnp.float32)
    big = jnp.einsum("ab,ifj->iafbj", eye8, M).reshape(nl, 8 * de, 8 * h)
    bigc = jnp.tile(cvec, (1, 8)).reshape(nl, 1, 8 * h)
    ea_packed = edge_attr.reshape(e // 8, 8 * de)

    hcur = h0
    for i in range(nl):
        e_msg = _emsg(ea_packed, big[i], bigc[i]).reshape(e, h)
        agg = _sc_agg(hcur, e_msg, src, dst)
        z1, stats = _mlp1(hcur, agg, W1[i], b1[i])
        hcur = _mlp2(z1, stats, gamma[i], beta[i], W2[i], b2[i])

    batch3 = batch.reshape(10, 1, n // 10)
    out_Wp = jnp.pad(out_W, ((0, 0), (0, h - c)))
    out_bp = jnp.pad(out_b, (0, h - c)).reshape(1, h)
    return _pool(hcur, batch3, out_Wp, out_bp)[:, :c]


# speedup vs baseline: 3.8765x; 1.6417x over previous
import functools

import jax
import jax.numpy as jnp
from jax import lax
from jax.experimental import pallas as pl
from jax.experimental.pallas import tpu as pltpu
from jax.experimental.pallas import tpu_sc as plsc

NC = 2
NS = 16
NW = NC * NS
EB = 80
GOUT = 64


def _h0_body(x_ref, w_ref, b_ref, o_ref):
    o_ref[...] = (
        jnp.dot(x_ref[...], w_ref[...], preferred_element_type=jnp.float32)
        + b_ref[...]
    )


def _h0(x, node_W, node_b):
    n, d = x.shape
    h = node_W.shape[1]
    nb = 10
    return pl.pallas_call(
        _h0_body,
        grid=(nb,),
        in_specs=[
            pl.BlockSpec((n // nb, d), lambda i: (i, 0)),
            pl.BlockSpec((d, h), lambda i: (0, 0)),
            pl.BlockSpec((1, h), lambda i: (0, 0)),
        ],
        out_specs=pl.BlockSpec((n // nb, h), lambda i: (i, 0)),
        out_shape=jax.ShapeDtypeStruct((n, h), jnp.float32),
    )(x, node_W, node_b.reshape(1, h))


def _fold_body(ew_ref, eb_ref, lw_ref, lb_ref, m_ref, c_ref):
    nl = lw_ref.shape[0]
    for i in range(nl):
        m_ref[i] = jnp.dot(ew_ref[...], lw_ref[i], preferred_element_type=jnp.float32)
        c_ref[i] = (
            jnp.dot(eb_ref[...], lw_ref[i], preferred_element_type=jnp.float32)[0]
            + lb_ref[i]
        )


def _fold(edge_W, edge_b, lin_W, lin_b):
    de, h = edge_W.shape
    nl = lin_W.shape[0]
    return pl.pallas_call(
        _fold_body,
        out_shape=[
            jax.ShapeDtypeStruct((nl, de, h), jnp.float32),
            jax.ShapeDtypeStruct((nl, h), jnp.float32),
        ],
    )(edge_W, edge_b.reshape(1, h), lin_W, lin_b)


def _emsg_body(ea_ref, big_ref, c_ref, o_ref):
    o_ref[...] = (
        jnp.dot(ea_ref[...], big_ref[...], preferred_element_type=jnp.float32)
        + c_ref[...]
    )


def _emsg(ea_packed, bigM, bigc):
    e8, k = ea_packed.shape
    kh = bigM.shape[1]
    nb = 50
    return pl.pallas_call(
        _emsg_body,
        grid=(nb,),
        in_specs=[
            pl.BlockSpec((e8 // nb, k), lambda i: (i, 0)),
            pl.BlockSpec((k, kh), lambda i: (0, 0)),
            pl.BlockSpec((1, kh), lambda i: (0, 0)),
        ],
        out_specs=pl.BlockSpec((e8 // nb, kh), lambda i: (i, 0)),
        out_shape=jax.ShapeDtypeStruct((e8, kh), jnp.float32),
    )(ea_packed, bigM, bigc)


def _sc_body(n, h, epw, nblk, rpt,
             h_hbm, e_hbm, src_hbm, dst_hbm, out_hbm,
             sidx, didx, ebuf, gbufA, gbufB, agg,
             isem, esem, dsemA, dsemB, gsemA, gsemB):
    c = lax.axis_index("c")
    s = lax.axis_index("s")
    wid = c * NS + s
    nv = h // 16
    tail = n - NS * rpt
    ebase = wid * epw

    pltpu.async_copy(src_hbm.at[wid], sidx, isem)

    zero = jnp.zeros((16,), jnp.float32)

    def zrow(r, carry):
        for k in range(nv):
            gbufA[r, pl.ds(k * 16, 16)] = zero
        return carry

    lax.fori_loop(0, EB, zrow, 0)
    row0 = s * rpt
    nfull, rem = divmod(rpt, EB)
    for t in range(nfull):
        pltpu.sync_copy(gbufA, agg.at[pl.ds(row0 + t * EB, EB)])
    if rem:
        pltpu.sync_copy(gbufA.at[pl.ds(0, rem)],
                        agg.at[pl.ds(row0 + nfull * EB, rem)])
    if tail:
        @pl.when(s == 0)
        def _():
            pltpu.sync_copy(gbufA.at[pl.ds(0, tail)], agg.at[pl.ds(NS * rpt, tail)])
    plsc.subcore_barrier()
    pltpu.make_async_copy(src_hbm.at[wid], sidx, isem).wait()

    def start_g(j, gbuf, gsem):
        idx = sidx.at[pl.ds(pl.multiple_of(j * EB, 8), EB)]
        pltpu.async_copy(h_hbm.at[idx], gbuf, gsem)

    def wait_g(gbuf, gsem):
        pltpu.make_async_copy(h_hbm.at[sidx.at[pl.ds(0, EB)]], gbuf, gsem).wait()

    def start_e(j):
        pltpu.async_copy(e_hbm.at[pl.ds(ebase + j * EB, EB)], ebuf, esem)

    def wait_e():
        pltpu.make_async_copy(e_hbm.at[pl.ds(ebase, EB)], ebuf, esem).wait()

    def start_d(j, p, dsem):
        pltpu.async_copy(dst_hbm.at[wid, j], didx.at[p], dsem)

    def wait_d(dsem):
        pltpu.make_async_copy(dst_hbm.at[0, 0], didx.at[0], dsem).wait()

    def relu(gbuf):
        def relu_row(r, carry2):
            for u in range(2):
                for k in range(nv):
                    sl = pl.ds(k * 16, 16)
                    rr = 2 * r + u
                    gbuf[rr, sl] = jnp.maximum(ebuf[rr, sl] + gbuf[rr, sl], 0.0)
            return carry2

        lax.fori_loop(0, EB // 2, relu_row, 0)

    def scatter(gbuf, p):
        pltpu.sync_copy(gbuf, agg.at[didx.at[p]], add=True)

    start_d(0, 0, dsemA)
    start_d(1, 1, dsemB)
    start_g(0, gbufA, gsemA)
    start_g(1, gbufB, gsemB)
    start_e(0)

    niter = (nblk - 1) // 2

    def step(jj, carry):
        j0 = 2 * jj

        wait_e()
        wait_g(gbufA, gsemA)
        relu(gbufA)
        start_e(j0 + 1)
        wait_d(dsemA)
        scatter(gbufA, 0)
        start_g(j0 + 2, gbufA, gsemA)
        start_d(j0 + 2, 0, dsemA)

        wait_e()
        wait_g(gbufB, gsemB)
        relu(gbufB)
        start_e(j0 + 2)
        wait_d(dsemB)
        scatter(gbufB, 1)

        @pl.when(jj < niter - 1)
        def _():
            start_g(j0 + 3, gbufB, gsemB)
            start_d(j0 + 3, 1, dsemB)

        return carry

    lax.fori_loop(0, niter, step, 0)

    for j in range(2 * niter, nblk):
        wait_e()
        wait_g(gbufA, gsemA)
        relu(gbufA)
        wait_d(dsemA)
        scatter(gbufA, 0)

    plsc.subcore_barrier()
    pltpu.sync_copy(agg.at[pl.ds(row0, rpt)], out_hbm.at[c, pl.ds(row0, rpt)])
    if tail:
        @pl.when(s == 0)
        def _():
            pltpu.sync_copy(agg.at[pl.ds(NS * rpt, tail)],
                            out_hbm.at[c, pl.ds(NS * rpt, tail)])


def _sc_agg(h_nodes, e_msg, src2, dst3):
    n, h = h_nodes.shape
    epw = src2.shape[1]
    nblk = epw // EB
    rpt = (n // NS) // 8 * 8

    mesh = plsc.VectorSubcoreMesh(core_axis_name="c", subcore_axis_name="s")
    body = functools.partial(_sc_body, n, h, epw, nblk, rpt)
    return pl.kernel(
        body,
        out_type=jax.ShapeDtypeStruct((NC, n, h), jnp.float32),
        mesh=mesh,
        scratch_types=[
            pltpu.VMEM((epw,), jnp.int32),
            pltpu.VMEM((2, EB), jnp.int32),
            pltpu.VMEM((EB, h), jnp.float32),
            pltpu.VMEM((EB, h), jnp.float32),
            pltpu.VMEM((EB, h), jnp.float32),
            pltpu.VMEM_SHARED((n, h), jnp.float32),
            pltpu.SemaphoreType.DMA,
            pltpu.SemaphoreType.DMA,
            pltpu.SemaphoreType.DMA,
            pltpu.SemaphoreType.DMA,
            pltpu.SemaphoreType.DMA,
            pltpu.SemaphoreType.DMA,
        ],
    )(h_nodes, e_msg, src2, dst3)


def _mlp1_body(h_ref, agg_ref, w_ref, b_ref, z_ref, st_ref):
    i = pl.program_id(0)
    z = h_ref[...] + agg_ref[0] + agg_ref[1]
    z1 = jnp.dot(z, w_ref[...], preferred_element_type=jnp.float32) + b_ref[...]
    z_ref[...] = z1
    ps = jnp.sum(z1, axis=0, keepdims=True)
    pss = jnp.sum(z1 * z1, axis=0, keepdims=True)
    blk = jnp.concatenate([ps, pss], axis=0)

    @pl.when(i == 0)
    def _():
        st_ref[...] = blk

    @pl.when(i != 0)
    def _():
        st_ref[...] = st_ref[...] + blk


def _mlp1(h_nodes, agg, W1, b1):
    n, h = h_nodes.shape
    nb = 10
    return pl.pallas_call(
        _mlp1_body,
        grid=(nb,),
        in_specs=[
            pl.BlockSpec((n // nb, h), lambda i: (i, 0)),
            pl.BlockSpec((NC, n // nb, h), lambda i: (0, i, 0)),
            pl.BlockSpec((h, h), lambda i: (0, 0)),
            pl.BlockSpec((1, h), lambda i: (0, 0)),
        ],
        out_specs=[
            pl.BlockSpec((n // nb, h), lambda i: (i, 0)),
            pl.BlockSpec((2, h), lambda i: (0, 0)),
        ],
        out_shape=[
            jax.ShapeDtypeStruct((n, h), jnp.float32),
            jax.ShapeDtypeStruct((2, h), jnp.float32),
        ],
    )(h_nodes, agg, W1, b1.reshape(1, h))


def _mlp2_body(n, z_ref, st_ref, g_ref, be_ref, w_ref, b_ref, o_ref):
    mean = st_ref[0] / n
    var = st_ref[1] / n - mean * mean
    inv = g_ref[0] * lax.rsqrt(var + 1e-5)
    xn = (z_ref[...] - mean) * inv + be_ref[0]
    a = jnp.maximum(xn, 0.0)
    z2 = jnp.dot(a, w_ref[...], preferred_element_type=jnp.float32) + b_ref[...]
    o_ref[...] = jnp.maximum(z2, 0.0)


def _mlp2(z1, stats, gamma, beta, W2, b2):
    n, h = z1.shape
    nb = 10
    body = functools.partial(_mlp2_body, float(n))
    return pl.pallas_call(
        body,
        grid=(nb,),
        in_specs=[
            pl.BlockSpec((n // nb, h), lambda i: (i, 0)),
            pl.BlockSpec((2, h), lambda i: (0, 0)),
            pl.BlockSpec((1, h), lambda i: (0, 0)),
            pl.BlockSpec((1, h), lambda i: (0, 0)),
            pl.BlockSpec((h, h), lambda i: (0, 0)),
            pl.BlockSpec((1, h), lambda i: (0, 0)),
        ],
        out_specs=pl.BlockSpec((n // nb, h), lambda i: (i, 0)),
        out_shape=jax.ShapeDtypeStruct((n, h), jnp.float32),
    )(z1, stats, gamma.reshape(1, h), beta.reshape(1, h), W2, b2.reshape(1, h))


def _pool_body(nb, h_ref, bat_ref, w_ref, b_ref, o_ref, acc):
    i = pl.program_id(0)
    b = bat_ref[0, 0, :]
    rows = b.shape[0]
    oh = (
        lax.broadcasted_iota(jnp.int32, (GOUT, rows), 0) == b[None, :]
    ).astype(jnp.float32)
    part = jnp.dot(oh, h_ref[...], preferred_element_type=jnp.float32)

    @pl.when(i == 0)
    def _():
        acc[...] = part

    @pl.when(i != 0)
    def _():
        acc[...] = acc[...] + part

    @pl.when(i == nb - 1)
    def _():
        o_ref[...] = (
            jnp.dot(acc[...], w_ref[...], preferred_element_type=jnp.float32)
            + b_ref[...]
        )


def _pool(h_nodes, batch3, out_Wp, out_bp):
    n, h = h_nodes.shape
    nb = 10
    cw = out_Wp.shape[1]
    body = functools.partial(_pool_body, nb)
    return pl.pallas_call(
        body,
        grid=(nb,),
        in_specs=[
            pl.BlockSpec((n // nb, h), lambda i: (i, 0)),
            pl.BlockSpec((1, 1, n // nb), lambda i: (i, 0, 0)),
            pl.BlockSpec((h, cw), lambda i: (0, 0)),
            pl.BlockSpec((1, cw), lambda i: (0, 0)),
        ],
        out_specs=pl.BlockSpec((GOUT, cw), lambda i: (0, 0)),
        out_shape=jax.ShapeDtypeStruct((GOUT, cw), jnp.float32),
        scratch_shapes=[pltpu.VMEM((GOUT, h), jnp.float32)],
    )(h_nodes, batch3, out_Wp, out_bp)


def kernel(x, edge_index, batch, edge_attr, node_W, node_b, edge_W, edge_b,
           lin_W, lin_b, W1, b1, gamma, beta, W2, b2, out_W, out_b):
    n, d = x.shape
    e = edge_index.shape[1]
    h = node_W.shape[1]
    de = edge_attr.shape[1]
    nl = lin_W.shape[0]
    c = out_W.shape[1]

    epw = e // NW
    src2 = edge_index[0].reshape(NW, epw)
    dst3 = edge_index[1].reshape(NW, epw // EB, EB)

    h0 = _h0(x, node_W, node_b)
    M, cvec = _fold(edge_W, edge_b, lin_W, lin_b)

    eye8 = jnp.eye(8, dtype=jnp.float32)
    big = jnp.einsum("ab,ifj->iafbj", eye8, M).reshape(nl, 8 * de, 8 * h)
    bigc = jnp.tile(cvec, (1, 8)).reshape(nl, 1, 8 * h)
    ea_packed = edge_attr.reshape(e // 8, 8 * de)

    hcur = h0
    for i in range(nl):
        e_msg = _emsg(ea_packed, big[i], bigc[i]).reshape(e, h)
        agg = _sc_agg(hcur, e_msg, src2, dst3)
        z1, stats = _mlp1(hcur, agg, W1[i], b1[i])
        hcur = _mlp2(z1, stats, gamma[i], beta[i], W2[i], b2[i])

    batch3 = batch.reshape(10, 1, n // 10)
    out_Wp = jnp.pad(out_W, ((0, 0), (0, h - c)))
    out_bp = jnp.pad(out_b, (0, h - c)).reshape(1, h)
    return _pool(hcur, batch3, out_Wp, out_bp)[:, :c]


# speedup vs baseline: 3.9770x; 1.0259x over previous
import functools

import jax
import jax.numpy as jnp
from jax import lax
from jax.experimental import pallas as pl
from jax.experimental.pallas import tpu as pltpu
from jax.experimental.pallas import tpu_sc as plsc

NC = 2
NS = 16
NW = NC * NS
EB = 80
CH_BLKS = 64
GOUT = 64


def _h0_body(x_ref, w_ref, b_ref, o_ref):
    o_ref[...] = (
        jnp.dot(x_ref[...], w_ref[...], preferred_element_type=jnp.float32)
        + b_ref[...]
    )


def _h0(x, node_W, node_b):
    n, d = x.shape
    h = node_W.shape[1]
    nb = 10
    return pl.pallas_call(
        _h0_body,
        grid=(nb,),
        in_specs=[
            pl.BlockSpec((n // nb, d), lambda i: (i, 0)),
            pl.BlockSpec((d, h), lambda i: (0, 0)),
            pl.BlockSpec((1, h), lambda i: (0, 0)),
        ],
        out_specs=pl.BlockSpec((n // nb, h), lambda i: (i, 0)),
        out_shape=jax.ShapeDtypeStruct((n, h), jnp.float32),
    )(x, node_W, node_b.reshape(1, h))


def _fold_body(ew_ref, eb_ref, lw_ref, lb_ref, m_ref, c_ref):
    nl = lw_ref.shape[0]
    for i in range(nl):
        m_ref[i] = jnp.dot(ew_ref[...], lw_ref[i], preferred_element_type=jnp.float32)
        c_ref[i] = (
            jnp.dot(eb_ref[...], lw_ref[i], preferred_element_type=jnp.float32)[0]
            + lb_ref[i]
        )


def _fold(edge_W, edge_b, lin_W, lin_b):
    de, h = edge_W.shape
    nl = lin_W.shape[0]
    return pl.pallas_call(
        _fold_body,
        out_shape=[
            jax.ShapeDtypeStruct((nl, de, h), jnp.float32),
            jax.ShapeDtypeStruct((nl, h), jnp.float32),
        ],
    )(edge_W, edge_b.reshape(1, h), lin_W, lin_b)


def _emsg3_body(ea_ref, big_ref, c_ref, o0_ref, o1_ref, o2_ref):
    for i, o_ref in enumerate((o0_ref, o1_ref, o2_ref)):
        o_ref[...] = (
            jnp.dot(ea_ref[...], big_ref[i], preferred_element_type=jnp.float32)
            + c_ref[i]
        )


def _emsg3(ea_packed, bigM, bigc):
    e8, k = ea_packed.shape
    nl, _, kh = bigM.shape
    nb = 50
    out = jax.ShapeDtypeStruct((e8, kh), jnp.float32)
    return pl.pallas_call(
        _emsg3_body,
        grid=(nb,),
        in_specs=[
            pl.BlockSpec((e8 // nb, k), lambda i: (i, 0)),
            pl.BlockSpec((nl, k, kh), lambda i: (0, 0, 0)),
            pl.BlockSpec((nl, 1, kh), lambda i: (0, 0, 0)),
        ],
        out_specs=[pl.BlockSpec((e8 // nb, kh), lambda i: (i, 0))] * 3,
        out_shape=[out, out, out],
    )(ea_packed, bigM, bigc)


def _sc_body(n, h, epw, nblk, rpt,
             h_hbm, e_hbm, src_hbm, dst_hbm, out_hbm,
             sidx, didx, ebufA, ebufB, gbufA, gbufB, agg,
             isem, esemA, esemB, dsemA, dsemB, gsemA, gsemB, ssemA, ssemB):
    c = lax.axis_index("c")
    s = lax.axis_index("s")
    wid = c * NS + s
    nv = h // 16
    hb = EB // 2
    tail = n - NS * rpt
    ebase = wid * epw

    pltpu.async_copy(src_hbm.at[wid], sidx, isem)

    zero = jnp.zeros((16,), jnp.float32)

    def zrow(r, carry):
        for k in range(nv):
            gbufA[r, pl.ds(k * 16, 16)] = zero
        return carry

    lax.fori_loop(0, EB, zrow, 0)
    row0 = s * rpt
    nfull, rem = divmod(rpt, EB)
    for t in range(nfull):
        pltpu.sync_copy(gbufA, agg.at[pl.ds(row0 + t * EB, EB)])
    if rem:
        pltpu.sync_copy(gbufA.at[pl.ds(0, rem)],
                        agg.at[pl.ds(row0 + nfull * EB, rem)])
    if tail:
        @pl.when(s == 0)
        def _():
            pltpu.sync_copy(gbufA.at[pl.ds(0, tail)], agg.at[pl.ds(NS * rpt, tail)])
    plsc.subcore_barrier()
    pltpu.make_async_copy(src_hbm.at[wid], sidx, isem).wait()

    def start_g(j, gbuf, gsem):
        idx = sidx.at[pl.ds(pl.multiple_of(j * EB, 8), EB)]
        pltpu.async_copy(h_hbm.at[idx], gbuf, gsem)

    def wait_g(gbuf, gsem):
        pltpu.make_async_copy(h_hbm.at[sidx.at[pl.ds(0, EB)]], gbuf, gsem).wait()

    def start_e(j, half, ebuf, esem):
        off = pl.multiple_of(ebase + j * EB + half * hb, 8)
        pltpu.async_copy(e_hbm.at[pl.ds(off, hb)], ebuf, esem)

    def wait_e(ebuf, esem):
        pltpu.make_async_copy(e_hbm.at[pl.ds(0, hb)], ebuf, esem).wait()

    def start_d(j, p, dsem):
        pltpu.async_copy(dst_hbm.at[wid, j], didx.at[p], dsem)

    def wait_d(dsem):
        pltpu.make_async_copy(dst_hbm.at[0, 0], didx.at[0], dsem).wait()

    def relu_half(ebuf, gbuf, base):
        def relu_row(r, carry2):
            for u in range(2):
                for k in range(nv):
                    sl = pl.ds(k * 16, 16)
                    rr = 2 * r + u
                    gbuf[base + rr, sl] = jnp.maximum(
                        ebuf[rr, sl] + gbuf[base + rr, sl], 0.0)
            return carry2

        lax.fori_loop(0, hb // 2, relu_row, 0)

    def start_s(gbuf, p, ssem):
        pltpu.async_copy(gbuf, agg.at[didx.at[p]], ssem, add=True)

    def wait_s(gbuf, ssem):
        pltpu.make_async_copy(gbuf, agg.at[didx.at[0]], ssem).wait()

    def process(j, gbuf, gsem, dsem, p, ssem):
        wait_g(gbuf, gsem)
        wait_e(ebufA, esemA)
        relu_half(ebufA, gbuf, 0)
        start_e(j + 1, 0, ebufA, esemA)
        wait_e(ebufB, esemB)
        relu_half(ebufB, gbuf, hb)
        start_e(j + 1, 1, ebufB, esemB)
        wait_d(dsem)
        start_s(gbuf, p, ssem)

    start_d(0, 0, dsemA)
    start_d(1, 1, dsemB)
    start_g(0, gbufA, gsemA)
    start_g(1, gbufB, gsemB)
    start_e(0, 0, ebufA, esemA)
    start_e(0, 1, ebufB, esemB)

    niter = (nblk - 1) // 2

    def step(jj, carry):
        j0 = 2 * jj

        process(j0, gbufA, gsemA, dsemA, 0, ssemA)
        process(j0 + 1, gbufB, gsemB, dsemB, 1, ssemB)

        wait_s(gbufA, ssemA)
        start_g(j0 + 2, gbufA, gsemA)
        start_d(j0 + 2, 0, dsemA)

        wait_s(gbufB, ssemB)

        @pl.when(jj < niter - 1)
        def _():
            start_g(j0 + 3, gbufB, gsemB)
            start_d(j0 + 3, 1, dsemB)

        return carry

    lax.fori_loop(0, niter, step, 0)

    for j in range(2 * niter, nblk):
        wait_g(gbufA, gsemA)
        wait_e(ebufA, esemA)
        relu_half(ebufA, gbufA, 0)
        wait_e(ebufB, esemB)
        relu_half(ebufB, gbufA, hb)
        wait_d(dsemA)
        pltpu.sync_copy(gbufA, agg.at[didx.at[0]], add=True)

    plsc.subcore_barrier()
    pltpu.sync_copy(agg.at[pl.ds(row0, rpt)], out_hbm.at[c, pl.ds(row0, rpt)])
    if tail:
        @pl.when(s == 0)
        def _():
            pltpu.sync_copy(agg.at[pl.ds(NS * rpt, tail)],
                            out_hbm.at[c, pl.ds(NS * rpt, tail)])


def _sc_agg(h_nodes, e_msg, src2, dst3):
    n, h = h_nodes.shape
    epw = src2.shape[1]
    nblk = epw // EB
    rpt = (n // NS) // 8 * 8

    mesh = plsc.VectorSubcoreMesh(core_axis_name="c", subcore_axis_name="s")
    body = functools.partial(_sc_body, n, h, epw, nblk, rpt)
    dma = pltpu.SemaphoreType.DMA
    return pl.kernel(
        body,
        out_type=jax.ShapeDtypeStruct((NC, n, h), jnp.float32),
        mesh=mesh,
        scratch_types=[
            pltpu.VMEM((epw,), jnp.int32),
            pltpu.VMEM((2, EB), jnp.int32),
            pltpu.VMEM((EB // 2, h), jnp.float32),
            pltpu.VMEM((EB // 2, h), jnp.float32),
            pltpu.VMEM((EB, h), jnp.float32),
            pltpu.VMEM((EB, h), jnp.float32),
            pltpu.VMEM_SHARED((n, h), jnp.float32),
            dma, dma, dma, dma, dma, dma, dma, dma, dma,
        ],
    )(h_nodes, e_msg, src2, dst3)


def _mlp_body(h_ref, agg_ref, w1_ref, b1_ref, g_ref, be_ref, w2_ref, b2_ref,
              o_ref):
    z = h_ref[...] + agg_ref[0] + agg_ref[1]
    z1 = jnp.dot(z, w1_ref[...], preferred_element_type=jnp.float32) + b1_ref[...]
    mean = jnp.mean(z1, axis=0, keepdims=True)
    var = jnp.mean(z1 * z1, axis=0, keepdims=True) - mean * mean
    xn = (z1 - mean) * (g_ref[...] * lax.rsqrt(var + 1e-5)) + be_ref[...]
    a = jnp.maximum(xn, 0.0)
    z2 = jnp.dot(a, w2_ref[...], preferred_element_type=jnp.float32) + b2_ref[...]
    o_ref[...] = jnp.maximum(z2, 0.0)


def _mlp(h_nodes, agg, W1, b1, gamma, beta, W2, b2):
    n, h = h_nodes.shape
    return pl.pallas_call(
        _mlp_body,
        out_shape=jax.ShapeDtypeStruct((n, h), jnp.float32),
    )(h_nodes, agg, W1, b1.reshape(1, h), gamma.reshape(1, h),
      beta.reshape(1, h), W2, b2.reshape(1, h))


def _pool_body(nb, h_ref, bat_ref, w_ref, b_ref, o_ref, acc):
    i = pl.program_id(0)
    b = bat_ref[0, 0, :]
    rows = b.shape[0]
    oh = (
        lax.broadcasted_iota(jnp.int32, (GOUT, rows), 0) == b[None, :]
    ).astype(jnp.float32)
    part = jnp.dot(oh, h_ref[...], preferred_element_type=jnp.float32)

    @pl.when(i == 0)
    def _():
        acc[...] = part

    @pl.when(i != 0)
    def _():
        acc[...] = acc[...] + part

    @pl.when(i == nb - 1)
    def _():
        o_ref[...] = (
            jnp.dot(acc[...], w_ref[...], preferred_element_type=jnp.float32)
            + b_ref[...]
        )


def _pool(h_nodes, batch3, out_Wp, out_bp):
    n, h = h_nodes.shape
    nb = 10
    cw = out_Wp.shape[1]
    body = functools.partial(_pool_body, nb)
    return pl.pallas_call(
        body,
        grid=(nb,),
        in_specs=[
            pl.BlockSpec((n // nb, h), lambda i: (i, 0)),
            pl.BlockSpec((1, 1, n // nb), lambda i: (i, 0, 0)),
            pl.BlockSpec((h, cw), lambda i: (0, 0)),
            pl.BlockSpec((1, cw), lambda i: (0, 0)),
        ],
        out_specs=pl.BlockSpec((GOUT, cw), lambda i: (0, 0)),
        out_shape=jax.ShapeDtypeStruct((GOUT, cw), jnp.float32),
        scratch_shapes=[pltpu.VMEM((GOUT, h), jnp.float32)],
    )(h_nodes, batch3, out_Wp, out_bp)


def kernel(x, edge_index, batch, edge_attr, node_W, node_b, edge_W, edge_b,
           lin_W, lin_b, W1, b1, gamma, beta, W2, b2, out_W, out_b):
    n, d = x.shape
    e = edge_index.shape[1]
    h = node_W.shape[1]
    de = edge_attr.shape[1]
    nl = lin_W.shape[0]
    c = out_W.shape[1]

    h0 = _h0(x, node_W, node_b)
    M, cvec = _fold(edge_W, edge_b, lin_W, lin_b)

    eye8 = jnp.eye(8, dtype=jnp.float32)
    big = jnp.einsum("ab,ifj->iafbj", eye8, M).reshape(nl, 8 * de, 8 * h)
    bigc = jnp.tile(cvec, (1, 8)).reshape(nl, 1, 8 * h)
    ea_packed = edge_attr.reshape(e // 8, 8 * de)

    e_msgs = [em.reshape(e, h) for em in _emsg3(ea_packed, big, bigc)]

    epw = e // NW
    src2 = edge_index[0].reshape(NW, epw)
    dst3 = edge_index[1].reshape(NW, epw // EB, EB)

    hcur = h0
    for i in range(nl):
        agg = _sc_agg(hcur, e_msgs[i], src2, dst3)
        hcur = _mlp(hcur, agg, W1[i], b1[i], gamma[i], beta[i], W2[i], b2[i])

    batch3 = batch.reshape(10, 1, n // 10)
    out_Wp = jnp.pad(out_W, ((0, 0), (0, h - c)))
    out_bp = jnp.pad(out_b, (0, h - c)).reshape(1, h)
    return _pool(hcur, batch3, out_Wp, out_bp)[:, :c]
